# Initial kernel scaffold; baseline (speedup 1.0000x reference)
import jax
import jax.numpy as jnp
from jax.experimental import pallas as pl


def kernel(x, prompts):
    raise NotImplementedError("write your pallas kernel here")


# speedup vs baseline: 6.4322x; 6.4322x over previous
import functools

import jax
import jax.numpy as jnp
from jax.experimental import pallas as pl
from jax.experimental.pallas import tpu as pltpu

N = 100000
D = 128
K = 16
KMEANS_ITERS = 5
BLOCK = 5000
NB = N // BLOCK


def _body(x_ref, prompts_ref, out_ref, cent_ref, sums_ref, counts_ref):
    i = pl.program_id(0)
    b = pl.program_id(1)

    @pl.when((i == 0) & (b == 0))
    def _init():
        cent_ref[...] = x_ref[0:K, :]

    @pl.when(b == 0)
    def _reset():
        sums_ref[...] = jnp.zeros_like(sums_ref)
        counts_ref[...] = jnp.zeros_like(counts_ref)

    x = x_ref[...]
    cent = cent_ref[...]
    x_sq = jnp.sum(x * x, axis=1, keepdims=True)
    c_sq = jnp.sum(cent * cent, axis=1)[None, :]
    xc = jax.lax.dot_general(x.astype(jnp.bfloat16), cent.astype(jnp.bfloat16),
                             (((1,), (1,)), ((), ())),
                             preferred_element_type=jnp.float32)
    d = x_sq - 2.0 * xc + c_sq
    labels = jnp.argmin(d, axis=1).astype(jnp.int32)
    onehot = (labels[:, None] == jax.lax.broadcasted_iota(jnp.int32, (1, K), 1)
              ).astype(jnp.float32)

    @pl.when(i < KMEANS_ITERS)
    def _accum():
        sums_ref[...] += jax.lax.dot_general(
            onehot, x, (((0,), (0,)), ((), ())),
            preferred_element_type=jnp.float32,
            precision=jax.lax.Precision.HIGHEST)
        counts_ref[...] += jax.lax.dot_general(
            onehot, jnp.ones((BLOCK, D), jnp.float32), (((0,), (0,)), ((), ())),
            preferred_element_type=jnp.float32)

    @pl.when((i < KMEANS_ITERS) & (b == NB - 1))
    def _update():
        counts = counts_ref[...]
        new_c = sums_ref[...] / jnp.maximum(counts, 1.0)
        cent_ref[...] = jnp.where(counts > 0, new_c, cent_ref[...])

    @pl.when(i == KMEANS_ITERS)
    def _emit():
        gathered = jax.lax.dot_general(
            onehot, prompts_ref[...], (((1,), (0,)), ((), ())),
            preferred_element_type=jnp.float32,
            precision=jax.lax.Precision.HIGHEST)
        out_ref[...] = x + gathered


@functools.partial(jax.jit, static_argnames=("interpret",))
def kernel(x, prompts, interpret=False):
    return pl.pallas_call(
        _body,
        grid=(KMEANS_ITERS + 1, NB),
        in_specs=[
            pl.BlockSpec((BLOCK, D), lambda i, b: (b, 0)),
            pl.BlockSpec((K, D), lambda i, b: (0, 0)),
        ],
        out_specs=pl.BlockSpec(
            (BLOCK, D),
            lambda i, b: (jnp.where(i == KMEANS_ITERS, b, 0), 0)),
        out_shape=jax.ShapeDtypeStruct((N, D), jnp.float32),
        scratch_shapes=[
            pltpu.VMEM((K, D), jnp.float32),
            pltpu.VMEM((K, D), jnp.float32),
            pltpu.VMEM((K, D), jnp.float32),
        ],
        compiler_params=pltpu.CompilerParams(
            dimension_semantics=("arbitrary", "arbitrary")),
        interpret=interpret,
    )(x, prompts)
